# baseline (speedup 1.0000x reference)
import functools

import jax
import jax.numpy as jnp
from jax import lax
from jax.experimental import pallas as pl
from jax.experimental.pallas import tpu as pltpu
from jax.experimental.pallas import tpu_sc as plsc

_NC = 2
_NS = 16
_NW = _NC * _NS
_LANES = 16
_IDX_CHUNK = 128


def _make_sc_kernel(batch, dim):
    assert batch % (8 * _NW) == 0
    assert dim == 2 * _LANES
    b_per_w = batch // _NW
    n_chunks = b_per_w // _IDX_CHUNK
    mesh = plsc.VectorSubcoreMesh(core_axis_name="c", subcore_axis_name="s")

    @functools.partial(
        pl.kernel,
        mesh=mesh,
        compiler_params=pltpu.CompilerParams(use_tc_tiling_on_sc=False,
                                             needs_layout_passes=False),
        out_type=jax.ShapeDtypeStruct((batch,), jnp.float32),
        scratch_types=[
            pltpu.VMEM((b_per_w,), jnp.int32),
            pltpu.VMEM((b_per_w,), jnp.int32),
            pltpu.VMEM((b_per_w, dim), jnp.float32),
            pltpu.VMEM((b_per_w, dim), jnp.float32),
            pltpu.VMEM((b_per_w,), jnp.float32),
            pltpu.VMEM((_LANES * _LANES,), jnp.float32),
            pltpu.SemaphoreType.DMA,
            pltpu.SemaphoreType.DMA,
        ],
    )
    def sc_kernel(uid_hbm, iid_hbm, utab_hbm, itab_hbm, out_hbm,
                  uidx_v, iidx_v, urows_v, irows_v, out_v, stage_v,
                  usem, isem):
        wid = lax.axis_index("s") * _NC + lax.axis_index("c")
        base = wid * b_per_w

        pltpu.sync_copy(uid_hbm.at[pl.ds(base, b_per_w)], uidx_v)
        pltpu.sync_copy(iid_hbm.at[pl.ds(base, b_per_w)], iidx_v)

        for j in range(n_chunks):
            sl = pl.ds(j * _IDX_CHUNK, _IDX_CHUNK)
            pltpu.async_copy(utab_hbm.at[uidx_v.at[sl]], urows_v.at[sl], usem)
            pltpu.async_copy(itab_hbm.at[iidx_v.at[sl]], irows_v.at[sl], isem)
        for j in range(n_chunks):
            sl = pl.ds(j * _IDX_CHUNK, _IDX_CHUNK)
            pltpu.make_async_copy(utab_hbm.at[uidx_v.at[sl]], urows_v.at[sl],
                                  usem).wait()
            pltpu.make_async_copy(itab_hbm.at[iidx_v.at[sl]], irows_v.at[sl],
                                  isem).wait()

        lane_iota = lax.iota(jnp.int32, _LANES)
        col_base = lane_iota * _LANES

        def body(g, _):
            row0 = g * _LANES
            for rl in range(_LANES):
                u0 = urows_v[row0 + rl, pl.ds(0, _LANES)]
                i0 = irows_v[row0 + rl, pl.ds(0, _LANES)]
                u1 = urows_v[row0 + rl, pl.ds(_LANES, _LANES)]
                i1 = irows_v[row0 + rl, pl.ds(_LANES, _LANES)]
                stage_v[pl.ds(rl * _LANES, _LANES)] = u0 * i0 + u1 * i1
            acc = plsc.load_gather(stage_v, [col_base])
            for c in range(1, _LANES):
                acc = acc + plsc.load_gather(stage_v, [col_base + c])
            out_v[pl.ds(row0, _LANES)] = acc
            return 0

        lax.fori_loop(0, b_per_w // _LANES, body, 0)

        pltpu.sync_copy(out_v, out_hbm.at[pl.ds(base, b_per_w)])

    return sc_kernel


@jax.jit
def kernel(user_id, item_id, user_table, item_table):
    batch = user_id.shape[0]
    dim = user_table.shape[1]
    fn = _make_sc_kernel(batch, dim)
    return fn(user_id, item_id, user_table, item_table)
